# Initial kernel scaffold; baseline (speedup 1.0000x reference)
import jax
import jax.numpy as jnp
from jax.experimental import pallas as pl


def kernel(initial_embeddings, edge_index, etype, norm, basis, comp, loop_weight, h_bias, ffn_W, ffn_b, ln_gamma, ln_beta):
    raise NotImplementedError("write your pallas kernel here")


# speedup vs baseline: 11.6118x; 11.6118x over previous
import jax
import jax.numpy as jnp
from jax import lax
from jax.experimental import pallas as pl
from jax.experimental.pallas import tpu as pltpu
from jax.experimental.pallas import tpu_sc as plsc

_N = 10000
_E = 320000
_D = 128
_R = 8
_NB = 4

_NC = 2
_NS = 16
_NW = _NC * _NS
_CH = 128
_CHUNKS = -(-_E // (_NW * _CH))
_PER_TILE = _CHUNKS * _CH
_E_PAD = _PER_TILE * _NW
_N_PAD = 10240
_RPT = _N_PAD // _NS

_BN = 400
_G1 = _N // _BN


def _tc_pre_body(comp_ref, h_ref, basis_ref, loopw_ref, hw_ref, loopp_ref, w_s):
    @pl.when(pl.program_id(0) == 0)
    def _():
        for r in range(_R):
            w = comp_ref[r, 0] * basis_ref[0]
            for b in range(1, _NB):
                w = w + comp_ref[r, b] * basis_ref[b]
            w_s[r] = w

    h = h_ref[...]
    for r in range(_R):
        hw_ref[r] = jnp.dot(h, w_s[r], preferred_element_type=jnp.float32)
    loopp_ref[...] = jnp.dot(h, loopw_ref[...], preferred_element_type=jnp.float32)


def _tc_pre(comp, h, basis, loop_weight):
    return pl.pallas_call(
        _tc_pre_body,
        grid=(_G1,),
        in_specs=[
            pl.BlockSpec(memory_space=pltpu.SMEM),
            pl.BlockSpec((_BN, _D), lambda i: (i, 0)),
            pl.BlockSpec((_NB, _D, _D), lambda i: (0, 0, 0)),
            pl.BlockSpec((_D, _D), lambda i: (0, 0)),
        ],
        out_specs=[
            pl.BlockSpec((_R, _BN, _D), lambda i: (0, i, 0)),
            pl.BlockSpec((_BN, _D), lambda i: (i, 0)),
        ],
        out_shape=[
            jax.ShapeDtypeStruct((_R, _N, _D), jnp.float32),
            jax.ShapeDtypeStruct((_N, _D), jnp.float32),
        ],
        scratch_shapes=[pltpu.VMEM((_R, _D, _D), jnp.float32)],
    )(comp, h, basis, loop_weight)


def _sc_edge_body(src_hbm, et_hbm, dst_hbm, nrm_hbm, hw_hbm, out_hbm,
                  src_v, et_v, dst_v, nrm_v, gidx_v, rows_v, acc_sh, sem):
    c = lax.axis_index("c")
    s = lax.axis_index("s")
    wid = c * _NS + s

    zero16 = jnp.zeros((16,), jnp.float32)

    def _zrow(e, carry):
        for q in range(_D // 16):
            rows_v[e, pl.ds(q * 16, 16)] = zero16
        return carry

    lax.fori_loop(0, _CH, _zrow, 0)

    r0 = s * _RPT
    for k in range(_RPT // _CH):
        pltpu.sync_copy(rows_v, acc_sh.at[pl.ds(r0 + k * _CH, _CH)])
    plsc.subcore_barrier()

    def _chunk(g, carry):
        base = wid * _PER_TILE + g * _CH
        pltpu.sync_copy(src_hbm.at[pl.ds(base, _CH)], src_v)
        pltpu.sync_copy(et_hbm.at[pl.ds(base, _CH)], et_v)
        pltpu.sync_copy(dst_hbm.at[pl.ds(base, _CH)], dst_v)
        pltpu.sync_copy(nrm_hbm.at[pl.ds(base, _CH)], nrm_v)

        def _gix(i, cc):
            sl = pl.ds(i * 16, 16)
            gidx_v[sl] = et_v[sl] * _N + src_v[sl]
            return cc

        lax.fori_loop(0, _CH // 16, _gix, 0)
        pltpu.async_copy(hw_hbm.at[gidx_v], rows_v, sem).wait()

        def _scale(g, cc):
            nv = nrm_v[pl.ds(g * 16, 16)]
            for j in range(16):
                e = g * 16 + j
                nsp = nv.at[jnp.full((16,), j, jnp.int32)].get(
                    mode='promise_in_bounds')
                for q in range(_D // 16):
                    sl = pl.ds(q * 16, 16)
                    rows_v[e, sl] = rows_v[e, sl] * nsp
            return cc

        lax.fori_loop(0, _CH // 16, _scale, 0)
        pltpu.sync_copy(rows_v, acc_sh.at[dst_v], add=True)
        return carry

    lax.fori_loop(0, _CHUNKS, _chunk, 0)
    plsc.subcore_barrier()
    pltpu.sync_copy(acc_sh.at[pl.ds(r0, _RPT)], out_hbm.at[c, pl.ds(r0, _RPT)])


def _sc_edge(src_p, et_p, dst_p, nrm_p, hw_flat):
    k = pl.kernel(
        _sc_edge_body,
        out_type=jax.ShapeDtypeStruct((_NC, _N_PAD, _D), jnp.float32),
        mesh=plsc.VectorSubcoreMesh(core_axis_name="c", subcore_axis_name="s"),
        scratch_types=[
            pltpu.VMEM((_CH,), jnp.int32),
            pltpu.VMEM((_CH,), jnp.int32),
            pltpu.VMEM((_CH,), jnp.int32),
            pltpu.VMEM((_CH,), jnp.float32),
            pltpu.VMEM((_CH,), jnp.int32),
            pltpu.VMEM((_CH, _D), jnp.float32),
            pltpu.VMEM_SHARED((_N_PAD, _D), jnp.float32),
            pltpu.SemaphoreType.DMA,
        ],
    )
    return k(src_p, et_p, dst_p, nrm_p, hw_flat)


def _tc_post_body(p_ref, loopp_ref, h_ref, ffnw_ref, hb_ref, fb_ref,
                  lg_ref, lb_ref, out_ref):
    node = p_ref[0] + p_ref[1] + loopp_ref[...] + hb_ref[...]
    node = jnp.maximum(node, 0.0)
    y = lax.dot_general(node, ffnw_ref[...], (((1,), (1,)), ((), ())),
                        preferred_element_type=jnp.float32)
    y = y + fb_ref[...] + h_ref[...]
    mu = jnp.mean(y, axis=1, keepdims=True)
    d = y - mu
    var = jnp.mean(d * d, axis=1, keepdims=True)
    out_ref[...] = d * lax.rsqrt(var + 1e-8) * lg_ref[...] + lb_ref[...]


def _tc_post(partials, loopp, h, ffn_W, h_bias, ffn_b, ln_gamma, ln_beta):
    vec = pl.BlockSpec((1, _D), lambda i: (0, 0))
    return pl.pallas_call(
        _tc_post_body,
        grid=(_G1,),
        in_specs=[
            pl.BlockSpec((_NC, _BN, _D), lambda i: (0, i, 0)),
            pl.BlockSpec((_BN, _D), lambda i: (i, 0)),
            pl.BlockSpec((_BN, _D), lambda i: (i, 0)),
            pl.BlockSpec((_D, _D), lambda i: (0, 0)),
            vec, vec, vec, vec,
        ],
        out_specs=pl.BlockSpec((_BN, _D), lambda i: (i, 0)),
        out_shape=jax.ShapeDtypeStruct((_N, _D), jnp.float32),
    )(partials, loopp, h, ffn_W, h_bias, ffn_b, ln_gamma, ln_beta)


def kernel(initial_embeddings, edge_index, etype, norm, basis, comp,
           loop_weight, h_bias, ffn_W, ffn_b, ln_gamma, ln_beta):
    h = initial_embeddings
    pad = _E_PAD - _E
    src_p = jnp.pad(edge_index[0], (0, pad))
    et_p = jnp.pad(etype, (0, pad))
    dst_p = jnp.pad(edge_index[1], (0, pad))
    nrm_p = jnp.pad(norm[:, 0], (0, pad))

    hw, loopp = _tc_pre(comp, h, basis, loop_weight)
    partials = _sc_edge(src_p, et_p, dst_p, nrm_p, hw.reshape(_R * _N, _D))
    return _tc_post(partials, loopp, h, ffn_W,
                    h_bias.reshape(1, _D), ffn_b.reshape(1, _D),
                    ln_gamma.reshape(1, _D), ln_beta.reshape(1, _D))
